# initial kernel scaffold (unmeasured)
import jax
import jax.numpy as jnp
from jax import lax
from jax.experimental import pallas as pl
from jax.experimental.pallas import tpu as pltpu


def kernel(
    x,
):
    def body(*refs):
        pass

    out_shape = jax.ShapeDtypeStruct(..., jnp.float32)
    return pl.pallas_call(body, out_shape=out_shape)(...)


# baseline (device time: 361237 ns/iter reference)
import jax
import jax.numpy as jnp
from jax import lax
from jax.experimental import pallas as pl
from jax.experimental.pallas import tpu as pltpu

M, N = 4096, 2048
AXES = ("x", "y", "z")


def kernel(x):
    x = x.reshape(M, N)

    def body(x_ref, out_ref, recv_ref, send_sems, recv_sems):
        bits = [lax.axis_index(a) for a in AXES]
        partners = [
            tuple(1 - b if i == ax else b for i, b in enumerate(bits))
            for ax in range(3)
        ]

        barrier = pltpu.get_barrier_semaphore()
        for p in partners:
            pl.semaphore_signal(
                barrier, inc=1, device_id=p, device_id_type=pl.DeviceIdType.MESH
            )
        pl.semaphore_wait(barrier, 3)

        out_ref[...] = x_ref[...].astype(jnp.bfloat16)

        off = jnp.int32(0)
        cur = M
        for s in range(3):
            half = cur // 2
            b = bits[s]
            send_off = off + (1 - b) * half
            keep_off = off + b * half
            rdma = pltpu.make_async_remote_copy(
                src_ref=out_ref.at[pl.ds(send_off, half)],
                dst_ref=recv_ref.at[pl.ds(0, half)],
                send_sem=send_sems.at[s],
                recv_sem=recv_sems.at[s],
                device_id=partners[s],
                device_id_type=pl.DeviceIdType.MESH,
            )
            rdma.start()
            rdma.wait()
            out_ref[pl.ds(keep_off, half), :] = (
                out_ref[pl.ds(keep_off, half), :] + recv_ref[pl.ds(0, half), :]
            )
            off = keep_off
            cur = half

        for t in range(3):
            s = 3 + t
            ax = 2 - t
            b = bits[ax]
            rdma = pltpu.make_async_remote_copy(
                src_ref=out_ref.at[pl.ds(off, cur)],
                dst_ref=out_ref.at[pl.ds(off, cur)],
                send_sem=send_sems.at[s],
                recv_sem=recv_sems.at[s],
                device_id=partners[ax],
                device_id_type=pl.DeviceIdType.MESH,
            )
            rdma.start()
            rdma.wait()
            off = off - b * cur
            cur = cur * 2

    return pl.pallas_call(
        body,
        out_shape=jax.ShapeDtypeStruct((M, N), jnp.bfloat16),
        in_specs=[pl.BlockSpec(memory_space=pltpu.VMEM)],
        out_specs=pl.BlockSpec(memory_space=pltpu.VMEM),
        scratch_shapes=[
            pltpu.VMEM((M // 2, N), jnp.bfloat16),
            pltpu.SemaphoreType.DMA((6,)),
            pltpu.SemaphoreType.DMA((6,)),
        ],
        compiler_params=pltpu.CompilerParams(
            collective_id=0, vmem_limit_bytes=100 * 1024 * 1024
        ),
    )(x)


# device time: 156058 ns/iter; 2.3148x vs baseline; 2.3148x over previous
import jax
import jax.numpy as jnp
from jax import lax
from jax.experimental import pallas as pl
from jax.experimental.pallas import tpu as pltpu

M, N = 4096, 2048
AXES = ("x", "y", "z")
CHUNKS = (
    (0, 1408, (0, 1, 2)),
    (1408, 1408, (1, 2, 0)),
    (2816, 1280, (2, 0, 1)),
)
RECV_BASE = (0, 704, 1408)


def kernel(x):
    x = x.reshape(M, N)

    def body(x_ref, out_ref, recv_ref, send_sems, recv_sems):
        bits = [lax.axis_index(a) for a in AXES]
        partners = [
            tuple(1 - b if i == ax else b for i, b in enumerate(bits))
            for ax in range(3)
        ]

        barrier = pltpu.get_barrier_semaphore()
        for p in partners:
            pl.semaphore_signal(
                barrier, inc=1, device_id=p, device_id_type=pl.DeviceIdType.MESH
            )
        pl.semaphore_wait(barrier, 3)

        out_ref[...] = x_ref[...].astype(jnp.bfloat16)

        offs = [jnp.int32(base) for base, _, _ in CHUNKS]
        curs = [rows for _, rows, _ in CHUNKS]

        for s in range(3):
            rdmas = []
            for ci, (_, _, order) in enumerate(CHUNKS):
                half = curs[ci] // 2
                b = bits[order[s]]
                send_off = offs[ci] + (1 - b) * half
                rdma = pltpu.make_async_remote_copy(
                    src_ref=out_ref.at[pl.ds(send_off, half)],
                    dst_ref=recv_ref.at[pl.ds(RECV_BASE[ci], half)],
                    send_sem=send_sems.at[s * 3 + ci],
                    recv_sem=recv_sems.at[s * 3 + ci],
                    device_id=partners[order[s]],
                    device_id_type=pl.DeviceIdType.MESH,
                )
                rdma.start()
                rdmas.append(rdma)
            for ci, (_, _, order) in enumerate(CHUNKS):
                half = curs[ci] // 2
                b = bits[order[s]]
                keep_off = offs[ci] + b * half
                rdmas[ci].wait()
                out_ref[pl.ds(keep_off, half), :] = (
                    out_ref[pl.ds(keep_off, half), :]
                    + recv_ref[pl.ds(RECV_BASE[ci], half), :]
                )
                offs[ci] = keep_off
                curs[ci] = half

        for t in range(3):
            rdmas = []
            for ci, (_, _, order) in enumerate(CHUNKS):
                ax = order[2 - t]
                rdma = pltpu.make_async_remote_copy(
                    src_ref=out_ref.at[pl.ds(offs[ci], curs[ci])],
                    dst_ref=out_ref.at[pl.ds(offs[ci], curs[ci])],
                    send_sem=send_sems.at[9 + t * 3 + ci],
                    recv_sem=recv_sems.at[9 + t * 3 + ci],
                    device_id=partners[ax],
                    device_id_type=pl.DeviceIdType.MESH,
                )
                rdma.start()
                rdmas.append(rdma)
            for ci, (_, _, order) in enumerate(CHUNKS):
                b = bits[order[2 - t]]
                rdmas[ci].wait()
                offs[ci] = offs[ci] - b * curs[ci]
                curs[ci] = curs[ci] * 2

    return pl.pallas_call(
        body,
        out_shape=jax.ShapeDtypeStruct((M, N), jnp.bfloat16),
        in_specs=[pl.BlockSpec(memory_space=pltpu.VMEM)],
        out_specs=pl.BlockSpec(memory_space=pltpu.VMEM),
        scratch_shapes=[
            pltpu.VMEM((M // 2, N), jnp.bfloat16),
            pltpu.SemaphoreType.DMA((18,)),
            pltpu.SemaphoreType.DMA((18,)),
        ],
        compiler_params=pltpu.CompilerParams(
            collective_id=0, vmem_limit_bytes=100 * 1024 * 1024
        ),
    )(x)
